# baseline (speedup 1.0000x reference)
import functools

import jax
import jax.numpy as jnp
from jax import lax
from jax.experimental import pallas as pl
from jax.experimental.pallas import tpu as pltpu
from jax.experimental.pallas import tpu_sc as plsc

B = 16384
D = 64
NC = 2
NS = 16
NW = NC * NS
BPW = B // NW
CHUNK = 128
NCHUNK = BPW // CHUNK


def _sc_gather_body(uid_hbm, iid_hbm, u_mlp_hbm, i_mlp_hbm, u_gmf_hbm,
                    i_gmf_hbm, out_um, out_im, out_ug, out_ig,
                    idx_u, idx_i, rows, sem0, sem1):
    wid = lax.axis_index("s") * NC + lax.axis_index("c")
    base = wid * BPW

    for j in range(NCHUNK):
        pltpu.sync_copy(uid_hbm.at[pl.ds(base + j * CHUNK, CHUNK)], idx_u.at[j])
        pltpu.sync_copy(iid_hbm.at[pl.ds(base + j * CHUNK, CHUNK)], idx_i.at[j])

    steps = []
    for tbl, idx, out in ((u_mlp_hbm, idx_u, out_um),
                          (i_mlp_hbm, idx_i, out_im),
                          (u_gmf_hbm, idx_u, out_ug),
                          (i_gmf_hbm, idx_i, out_ig)):
        for j in range(NCHUNK):
            steps.append((tbl, idx, out, j))

    sems = (sem0, sem1)
    n = len(steps)
    descs = [None] * n

    tbl0, i0, _, j0 = steps[0]
    descs[0] = pltpu.async_copy(tbl0.at[i0.at[j0]], rows.at[0], sems[0])
    for k in range(n):
        buf = k % 2
        if k + 1 < n:
            tbl, idx, _, j = steps[k + 1]
            descs[k + 1] = pltpu.async_copy(
                tbl.at[idx.at[j]], rows.at[1 - buf], sems[1 - buf])
        descs[k].wait()
        _, _, out, j = steps[k]
        pltpu.sync_copy(rows.at[buf], out.at[pl.ds(base + j * CHUNK, CHUNK)])


@jax.jit
def _sc_gather(user_id, item_id, u_mlp, i_mlp, u_gmf, i_gmf):
    mesh = plsc.VectorSubcoreMesh(core_axis_name="c", subcore_axis_name="s")
    out = jax.ShapeDtypeStruct((B, D), jnp.float32)
    f = pl.kernel(
        _sc_gather_body,
        out_type=(out, out, out, out),
        mesh=mesh,
        scratch_types=[
            pltpu.VMEM((NCHUNK, CHUNK), jnp.int32),
            pltpu.VMEM((NCHUNK, CHUNK), jnp.int32),
            pltpu.VMEM((2, CHUNK, D), jnp.float32),
            pltpu.SemaphoreType.DMA,
            pltpu.SemaphoreType.DMA,
        ],
        compiler_params=pltpu.CompilerParams(use_tc_tiling_on_sc=False),
    )
    return f(user_id, item_id, u_mlp, i_mlp, u_gmf, i_gmf)


def _mlp_body(u_ref, i_ref, ug_ref, ig_ref, w0_ref, b0_ref, w1_ref, b1_ref,
              w2_ref, b2_ref, wp_ref, out_ref):
    h = jnp.concatenate([u_ref[...], i_ref[...]], axis=1)
    h = jnp.maximum(
        jnp.dot(h, w0_ref[...], preferred_element_type=jnp.float32)
        + b0_ref[...], 0.0)
    h = jnp.maximum(
        jnp.dot(h, w1_ref[...], preferred_element_type=jnp.float32)
        + b1_ref[...], 0.0)
    h = jnp.maximum(
        jnp.dot(h, w2_ref[...], preferred_element_type=jnp.float32)
        + b2_ref[...], 0.0)
    gmf = ug_ref[...] * ig_ref[...]
    fusion = jnp.concatenate([gmf, h], axis=1)
    out_ref[...] = jnp.sum(fusion * wp_ref[...], axis=1)


BM = 2048


def _tc_mlp(u_rows, i_rows, ug_rows, ig_rows, W0, b0, W1, b1, W2, b2, Wp,
            interpret=False):
    grid = (B // BM,)
    row_spec = pl.BlockSpec((BM, D), lambda i: (i, 0))
    full = lambda shape: pl.BlockSpec(shape, lambda i: tuple(0 for _ in shape))
    return pl.pallas_call(
        _mlp_body,
        grid=grid,
        in_specs=[
            row_spec, row_spec, row_spec, row_spec,
            full(W0.shape), full((1, 256)),
            full(W1.shape), full((1, 128)),
            full(W2.shape), full((1, 64)),
            full((1, 128)),
        ],
        out_specs=pl.BlockSpec((BM,), lambda i: (i,)),
        out_shape=jax.ShapeDtypeStruct((B,), jnp.float32),
        interpret=interpret,
    )(u_rows, i_rows, ug_rows, ig_rows, W0, b0.reshape(1, -1),
      W1, b1.reshape(1, -1), W2, b2.reshape(1, -1), Wp.reshape(1, -1))


def kernel(user_id, item_id, u_mlp, i_mlp, u_gmf, i_gmf,
           W0, b0, W1, b1, W2, b2, Wp):
    u_rows, i_rows, ug_rows, ig_rows = _sc_gather(
        user_id, item_id, u_mlp, i_mlp, u_gmf, i_gmf)
    return _tc_mlp(u_rows, i_rows, ug_rows, ig_rows,
                   W0, b0, W1, b1, W2, b2, Wp)


# speedup vs baseline: 1.1600x; 1.1600x over previous
import jax
import jax.numpy as jnp
from jax import lax
from jax.experimental import pallas as pl
from jax.experimental.pallas import tpu as pltpu
from jax.experimental.pallas import tpu_sc as plsc

B = 16384
D = 64
NC = 2
NS = 16
NW = NC * NS
BPW = B // NW
CHUNK = 128
NCHUNK = BPW // CHUNK


def _sc_gather_body(uid_hbm, iid_hbm, u_mlp_hbm, i_mlp_hbm, u_gmf_hbm,
                    i_gmf_hbm, out_mlp, out_gmf,
                    idx_u, idx_i, rows, sem0, sem1):
    wid = lax.axis_index("s") * NC + lax.axis_index("c")
    base = wid * BPW

    for j in range(NCHUNK):
        pltpu.sync_copy(uid_hbm.at[pl.ds(base + j * CHUNK, CHUNK)], idx_u.at[j])
        pltpu.sync_copy(iid_hbm.at[pl.ds(base + j * CHUNK, CHUNK)], idx_i.at[j])

    steps = []
    for tbl, idx, out, col in ((u_mlp_hbm, idx_u, out_mlp, 0),
                               (i_mlp_hbm, idx_i, out_mlp, D),
                               (u_gmf_hbm, idx_u, out_gmf, 0),
                               (i_gmf_hbm, idx_i, out_gmf, D)):
        for j in range(NCHUNK):
            steps.append((tbl, idx, out, col, j))

    sems = (sem0, sem1)
    n = len(steps)
    descs = [None] * n

    tbl0, i0, _, _, j0 = steps[0]
    descs[0] = pltpu.async_copy(tbl0.at[i0.at[j0]], rows.at[0], sems[0])
    for k in range(n):
        buf = k % 2
        if k + 1 < n:
            tbl, idx, _, _, j = steps[k + 1]
            descs[k + 1] = pltpu.async_copy(
                tbl.at[idx.at[j]], rows.at[1 - buf], sems[1 - buf])
        descs[k].wait()
        _, _, out, col, j = steps[k]
        pltpu.sync_copy(rows.at[buf],
                        out.at[pl.ds(base + j * CHUNK, CHUNK), pl.ds(col, D)])


@jax.jit
def _sc_gather(user_id, item_id, u_mlp, i_mlp, u_gmf, i_gmf):
    mesh = plsc.VectorSubcoreMesh(core_axis_name="c", subcore_axis_name="s")
    out = jax.ShapeDtypeStruct((B, 2 * D), jnp.float32)
    f = pl.kernel(
        _sc_gather_body,
        out_type=(out, out),
        mesh=mesh,
        scratch_types=[
            pltpu.VMEM((NCHUNK, CHUNK), jnp.int32),
            pltpu.VMEM((NCHUNK, CHUNK), jnp.int32),
            pltpu.VMEM((2, CHUNK, D), jnp.float32),
            pltpu.SemaphoreType.DMA,
            pltpu.SemaphoreType.DMA,
        ],
        compiler_params=pltpu.CompilerParams(use_tc_tiling_on_sc=False),
    )
    return f(user_id, item_id, u_mlp, i_mlp, u_gmf, i_gmf)


def _mlp_body(x1_ref, x2_ref, w0_ref, b0_ref, w1_ref, b1_ref,
              w2_ref, b2_ref, wp_ref, out_ref):
    h = jnp.maximum(
        jnp.dot(x1_ref[...], w0_ref[...], preferred_element_type=jnp.float32)
        + b0_ref[...], 0.0)
    h = jnp.maximum(
        jnp.dot(h, w1_ref[...], preferred_element_type=jnp.float32)
        + b1_ref[...], 0.0)
    h = jnp.maximum(
        jnp.dot(h, w2_ref[...], preferred_element_type=jnp.float32)
        + b2_ref[...], 0.0)
    x2 = x2_ref[...]
    gmf = x2[:, :D] * x2[:, D:]
    out_ref[...] = (
        jnp.dot(gmf, wp_ref[0:D, :], preferred_element_type=jnp.float32)
        + jnp.dot(h, wp_ref[D:2 * D, :], preferred_element_type=jnp.float32))


BM = 2048


def _tc_mlp(x1, x2, W0, b0, W1, b1, W2, b2, Wp, interpret=False):
    grid = (B // BM,)
    row_spec = pl.BlockSpec((BM, 2 * D), lambda i: (i, 0))
    full = lambda shape: pl.BlockSpec(shape, lambda i: tuple(0 for _ in shape))
    return pl.pallas_call(
        _mlp_body,
        grid=grid,
        in_specs=[
            row_spec, row_spec,
            full(W0.shape), full((1, 256)),
            full(W1.shape), full((1, 128)),
            full(W2.shape), full((1, 64)),
            full((128, 1)),
        ],
        out_specs=pl.BlockSpec((BM, 1), lambda i: (i, 0)),
        out_shape=jax.ShapeDtypeStruct((B, 1), jnp.float32),
        interpret=interpret,
    )(x1, x2, W0, b0.reshape(1, -1), W1, b1.reshape(1, -1),
      W2, b2.reshape(1, -1), Wp)


def kernel(user_id, item_id, u_mlp, i_mlp, u_gmf, i_gmf,
           W0, b0, W1, b1, W2, b2, Wp):
    x1, x2 = _sc_gather(user_id, item_id, u_mlp, i_mlp, u_gmf, i_gmf)
    return _tc_mlp(x1, x2, W0, b0, W1, b1, W2, b2, Wp).reshape(-1)
